# baseline (speedup 1.0000x reference)
import jax
import jax.numpy as jnp
from jax.experimental import pallas as pl
from jax.experimental.pallas import tpu as pltpu
from jax.experimental.pallas import tpu_sc as plsc

_EPS_BN = 1e-5
_MASKVAL = 1e10
_BIG = 1e30
_W = 16
_NUNITS = 32


def _stats_kernel(x1_ref, x2_ref, w1t_ref, w2t_ref, bgb1_ref, bgb2_ref,
                  h1_ref, feat_ref):
    def lin_bn_relu(x_ref, wt_ref, bgb_ref, o_ref):
        b = bgb_ref[0:1, :]
        gamma = bgb_ref[1:2, :]
        beta = bgb_ref[2:3, :]
        y = jnp.dot(x_ref[...], wt_ref[...],
                    preferred_element_type=jnp.float32) + b
        mu = jnp.mean(y, axis=0, keepdims=True)
        var = jnp.mean((y - mu) * (y - mu), axis=0, keepdims=True)
        s = gamma * jax.lax.rsqrt(var + _EPS_BN)
        o_ref[...] = jnp.maximum(y * s + (beta - mu * s), 0.0)

    lin_bn_relu(x1_ref, w1t_ref, bgb1_ref, h1_ref)
    lin_bn_relu(x2_ref, w2t_ref, bgb2_ref, feat_ref)


def _select_kernel(p1_ref, b1_ref, p2t_ref, b2_ref, idx_ref, wexp_ref):
    blk = p1_ref.shape[0]
    n2 = p2t_ref.shape[1]

    p1 = p1_ref[...]
    p2t = p2t_ref[...]

    dot = jnp.dot(p1, p2t, preferred_element_type=jnp.float32)
    sq1 = (p1[:, 0:1] * p1[:, 0:1] + p1[:, 1:2] * p1[:, 1:2]) \
        + p1[:, 2:3] * p1[:, 2:3]
    sq2 = (p2t[0:1, :] * p2t[0:1, :] + p2t[1:2, :] * p2t[1:2, :]) \
        + p2t[2:3, :] * p2t[2:3, :]
    d2 = sq1 + sq2 - 2.0 * dot
    same = b1_ref[...] == b2_ref[...]
    d2m = jnp.where(same, d2, _MASKVAL)

    d2e = None
    for c in range(3):
        diff = p1[:, c:c + 1] - p2t[c:c + 1, :]
        sq = diff * diff
        d2e = sq if d2e is None else d2e + sq

    iota = jax.lax.broadcasted_iota(jnp.int32, (blk, n2), 1)
    a = d2m
    msel = []
    sels = []
    idxs = []
    for k in range(3):
        m = jnp.min(a, axis=1, keepdims=True)
        sel = a == m
        i = jnp.min(jnp.where(sel, iota, n2), axis=1, keepdims=True)
        msel.append(m)
        sels.append(sel)
        idxs.append(jnp.minimum(i, n2 - 1))
        if k < 2:
            a = jnp.where(sel, _BIG, a)

    ws = []
    for k in range(3):
        mex = jnp.sum(jnp.where(sels[k], d2e, 0.0), axis=1, keepdims=True)
        w = jnp.where(msel[k] < 1e9,
                      1.0 / (jnp.sqrt(mex) + 1e-8), 0.0)
        ws.append(w)
    inv_norm = 1.0 / (ws[0] + ws[1] + ws[2])
    idx_ref[...] = jnp.concatenate(idxs, axis=1)
    wexp_ref[...] = jnp.concatenate(
        [jnp.broadcast_to(w * inv_norm, (blk, 16)) for w in ws]
        + [jnp.zeros((blk, 80), jnp.float32)], axis=1)


def _sc_combine(n1, c_out):
    rpu = n1 // _NUNITS
    mac = 128

    def run(h1, feat, i1, i2, i3, wexp):
        @pl.kernel(
            out_type=jax.ShapeDtypeStruct((n1, c_out), jnp.float32),
            mesh=plsc.VectorSubcoreMesh(core_axis_name="c",
                                        subcore_axis_name="s"),
            scratch_types=[
                pltpu.VMEM((_W, c_out), jnp.float32),
                pltpu.VMEM((mac, 128), jnp.float32),
                pltpu.VMEM((1, mac), jnp.int32),
                pltpu.VMEM((1, mac), jnp.int32),
                pltpu.VMEM((1, mac), jnp.int32),
                pltpu.VMEM((_W, c_out), jnp.float32),
                pltpu.VMEM((_W, c_out), jnp.float32),
                pltpu.VMEM((_W, c_out), jnp.float32),
                pltpu.VMEM((_W, c_out), jnp.float32),
            ],
        )
        def sck(h1_hbm, feat_hbm, i1_hbm, i2_hbm, i3_hbm, w_hbm, o_hbm,
                h1_v, w_v, i1_v, i2_v, i3_v, g1_v, g2_v, g3_v, o_v):
            c = jax.lax.axis_index("c")
            s = jax.lax.axis_index("s")
            u = c * 16 + s

            @pl.loop(0, rpu // mac)
            def _(wi):
                r0 = u * rpu + wi * mac
                pltpu.sync_copy(w_hbm.at[pl.ds(r0, mac), :], w_v)
                pltpu.sync_copy(i1_hbm.at[:, pl.ds(r0, mac)], i1_v)
                pltpu.sync_copy(i2_hbm.at[:, pl.ds(r0, mac)], i2_v)
                pltpu.sync_copy(i3_hbm.at[:, pl.ds(r0, mac)], i3_v)

                @pl.loop(0, mac // _W)
                def _(j):
                    g0 = r0 + j * _W
                    pltpu.sync_copy(h1_hbm.at[pl.ds(g0, _W), :], h1_v)
                    jw = j * _W
                    pltpu.sync_copy(
                        feat_hbm.at[i1_v.at[0, pl.ds(jw, _W)]], g1_v)
                    pltpu.sync_copy(
                        feat_hbm.at[i2_v.at[0, pl.ds(jw, _W)]], g2_v)
                    pltpu.sync_copy(
                        feat_hbm.at[i3_v.at[0, pl.ds(jw, _W)]], g3_v)

                    @pl.loop(0, _W)
                    def _(p):
                        @pl.loop(0, c_out, step=16)
                        def _(ch):
                            sp = (pl.ds(p, 1), pl.ds(ch, 16))
                            wp = pl.ds(jw + p, 1)
                            o_v.at[*sp][...] = (
                                h1_v.at[*sp][...]
                                + w_v.at[wp, pl.ds(0, 16)][...]
                                * g1_v.at[*sp][...]
                                + w_v.at[wp, pl.ds(16, 16)][...]
                                * g2_v.at[*sp][...]
                                + w_v.at[wp, pl.ds(32, 16)][...]
                                * g3_v.at[*sp][...])

                    pltpu.sync_copy(o_v, o_hbm.at[pl.ds(g0, _W), :])

        return sck(h1, feat, i1, i2, i3, wexp)

    return run


def kernel(x1, pos1, batch1, x2, pos2, batch2, W1, b1, gamma1, beta1,
           W2, b2, gamma2, beta2):
    n1, c_out = x1.shape
    n2, c_in = x2.shape

    b1f = batch1.astype(jnp.float32)[:, None]
    b2f = batch2.astype(jnp.float32)[None, :]
    p2t = pos2.T

    bgb1 = jnp.stack([b1, gamma1, beta1])
    bgb2 = jnp.stack([b2, gamma2, beta2])

    h1, feat = pl.pallas_call(
        _stats_kernel,
        out_shape=[
            jax.ShapeDtypeStruct((n1, c_out), jnp.float32),
            jax.ShapeDtypeStruct((n2, c_out), jnp.float32),
        ],
    )(x1, x2, W1.T, W2.T, bgb1, bgb2)

    blk = 512
    grid = n1 // blk
    idx, wexp = pl.pallas_call(
        _select_kernel,
        grid=(grid,),
        in_specs=[
            pl.BlockSpec((blk, 3), lambda i: (i, 0)),
            pl.BlockSpec((blk, 1), lambda i: (i, 0)),
            pl.BlockSpec((3, n2), lambda i: (0, 0)),
            pl.BlockSpec((1, n2), lambda i: (0, 0)),
        ],
        out_specs=[
            pl.BlockSpec((blk, 3), lambda i: (i, 0)),
            pl.BlockSpec((blk, 128), lambda i: (i, 0)),
        ],
        out_shape=[
            jax.ShapeDtypeStruct((n1, 3), jnp.int32),
            jax.ShapeDtypeStruct((n1, 128), jnp.float32),
        ],
    )(pos1, b1f, p2t, b2f)

    i1 = idx[:, 0].reshape(1, n1)
    i2 = idx[:, 1].reshape(1, n1)
    i3 = idx[:, 2].reshape(1, n1)
    x = _sc_combine(n1, c_out)(h1, feat, i1, i2, i3, wexp)
    return (x, pos1, batch1)


# speedup vs baseline: 1.9923x; 1.9923x over previous
import jax
import jax.numpy as jnp
from jax.experimental import pallas as pl

_EPS_BN = 1e-5
_MASKVAL = 1e10
_BIG = 1e30


def _stats_kernel(x1_ref, x2_ref, w1t_ref, w2t_ref, bgb1_ref, bgb2_ref,
                  h1_ref, feat_ref):
    def lin_bn_relu(x_ref, wt_ref, bgb_ref, o_ref):
        b = bgb_ref[0:1, :]
        gamma = bgb_ref[1:2, :]
        beta = bgb_ref[2:3, :]
        y = jnp.dot(x_ref[...], wt_ref[...],
                    preferred_element_type=jnp.float32) + b
        mu = jnp.mean(y, axis=0, keepdims=True)
        var = jnp.mean((y - mu) * (y - mu), axis=0, keepdims=True)
        s = gamma * jax.lax.rsqrt(var + _EPS_BN)
        o_ref[...] = jnp.maximum(y * s + (beta - mu * s), 0.0)

    lin_bn_relu(x1_ref, w1t_ref, bgb1_ref, h1_ref)
    lin_bn_relu(x2_ref, w2t_ref, bgb2_ref, feat_ref)


def _interp_kernel(h1_ref, p1_ref, b1_ref, p2t_ref, b2_ref, feat_ref,
                   out_ref):
    blk = h1_ref.shape[0]
    n2 = p2t_ref.shape[1]

    p1 = p1_ref[...]
    p2t = p2t_ref[...]

    dot = jnp.dot(p1, p2t, preferred_element_type=jnp.float32)
    sq1 = (p1[:, 0:1] * p1[:, 0:1] + p1[:, 1:2] * p1[:, 1:2]) \
        + p1[:, 2:3] * p1[:, 2:3]
    sq2 = (p2t[0:1, :] * p2t[0:1, :] + p2t[1:2, :] * p2t[1:2, :]) \
        + p2t[2:3, :] * p2t[2:3, :]
    d2 = sq1 + sq2 - 2.0 * dot
    same = b1_ref[...] == b2_ref[...]
    d2m = jnp.where(same, d2, _MASKVAL)

    d2e = None
    for c in range(3):
        diff = p1[:, c:c + 1] - p2t[c:c + 1, :]
        sq = diff * diff
        d2e = sq if d2e is None else d2e + sq

    a = d2m
    msel = []
    sels = []
    for k in range(3):
        m = jnp.min(a, axis=1, keepdims=True)
        sel = a == m
        msel.append(m)
        sels.append(sel)
        if k < 2:
            a = jnp.where(sel, _BIG, a)

    ws = []
    for k in range(3):
        mex = jnp.sum(jnp.where(sels[k], d2e, 0.0), axis=1, keepdims=True)
        w = jnp.where(msel[k] < 1e9,
                      1.0 / (jnp.sqrt(mex) + 1e-8), 0.0)
        ws.append(w)
    inv_norm = 1.0 / (ws[0] + ws[1] + ws[2])
    wmat = jnp.where(sels[0], ws[0] * inv_norm,
                     jnp.where(sels[1], ws[1] * inv_norm,
                               jnp.where(sels[2], ws[2] * inv_norm, 0.0)))
    nf = jnp.dot(wmat, feat_ref[...],
                 preferred_element_type=jnp.float32)
    out_ref[...] = h1_ref[...] + nf


def kernel(x1, pos1, batch1, x2, pos2, batch2, W1, b1, gamma1, beta1,
           W2, b2, gamma2, beta2):
    n1, c_out = x1.shape
    n2, c_in = x2.shape

    b1f = batch1.astype(jnp.float32)[:, None]
    b2f = batch2.astype(jnp.float32)[None, :]
    p2t = pos2.T

    bgb1 = jnp.stack([b1, gamma1, beta1])
    bgb2 = jnp.stack([b2, gamma2, beta2])

    h1, feat = pl.pallas_call(
        _stats_kernel,
        out_shape=[
            jax.ShapeDtypeStruct((n1, c_out), jnp.float32),
            jax.ShapeDtypeStruct((n2, c_out), jnp.float32),
        ],
    )(x1, x2, W1.T, W2.T, bgb1, bgb2)

    blk = 512
    grid = n1 // blk
    x = pl.pallas_call(
        _interp_kernel,
        grid=(grid,),
        in_specs=[
            pl.BlockSpec((blk, c_out), lambda i: (i, 0)),
            pl.BlockSpec((blk, 3), lambda i: (i, 0)),
            pl.BlockSpec((blk, 1), lambda i: (i, 0)),
            pl.BlockSpec((3, n2), lambda i: (0, 0)),
            pl.BlockSpec((1, n2), lambda i: (0, 0)),
            pl.BlockSpec((n2, c_out), lambda i: (0, 0)),
        ],
        out_specs=pl.BlockSpec((blk, c_out), lambda i: (i, 0)),
        out_shape=jax.ShapeDtypeStruct((n1, c_out), jnp.float32),
    )(h1, pos1, b1f, p2t, b2f, feat)
    return (x, pos1, batch1)
